# baseline (speedup 1.0000x reference)
import jax
import jax.numpy as jnp
from jax.experimental import pallas as pl
from jax.experimental.pallas import tpu as pltpu

_IN_CH = 96
_OUT_CH = 192
_K = 3
_STRIDE = 2
_PAD = 1
_TABLE = 16
_NH = 4
_M = 9
_U = 0.99
_R = 2.5
_B, _H, _W = 2, 224, 224
_HO = _WO = 112
_RB = 16
_NRB = _HO // _RB
_OCB = 64
_NOCB = _OUT_CH // _OCB
_LANE = 128
_HALO = 24


def _conv_body(wref, ee, eo, oe, oo, oref):
    def mm(w, xarr):
        return jax.lax.dot_general(
            w, xarr.reshape(_IN_CH, _RB * _LANE),
            (((1,), (0,)), ((), ())),
            preferred_element_type=jnp.float32)

    aee = ee[...]
    aeo = eo[...]
    ee0 = aee[:, 0:_RB]
    ee1 = aee[:, 1:_RB + 1]
    eo0 = aeo[:, 0:_RB]
    eo1 = aeo[:, 1:_RB + 1]
    xoe = oe[...]
    xoo = oo[...]
    w = wref[...]
    acc0 = (mm(w[0, 0], ee0) + mm(w[1, 0], xoe) + mm(w[2, 0], ee1)
            + mm(w[0, 1], eo0) + mm(w[1, 1], xoo) + mm(w[2, 1], eo1))
    acc1 = (mm(w[0, 2], ee0) + mm(w[1, 2], xoe) + mm(w[2, 2], ee1))
    acc1 = jnp.roll(acc1.reshape(_OCB, _RB, _LANE), -1, axis=2)
    oref[...] = acc0.reshape(_OCB, _RB, _LANE) + acc1


def _halo_rows(a):
    chunks = [jnp.pad(a[:, :, 16 * i:16 * i + 17, :],
                      ((0, 0), (0, 0), (0, _HALO - 17), (0, 0)))
              for i in range(_NRB)]
    return jnp.concatenate(chunks, axis=2)


def _pad_lanes(a):
    return jnp.pad(a, ((0, 0), (0, 0), (0, 0), (0, _LANE - a.shape[3])))


def _conv_pallas(wt, x):
    xp = jnp.pad(x, ((0, 0), (0, 0), (1, 1), (1, 1)))
    xe = xp[:, :, 0::2, :]
    xo = xp[:, :, 1::2, :]
    ee = _pad_lanes(_halo_rows(xe[:, :, :, 0::2]))
    eo = _pad_lanes(_halo_rows(xe[:, :, :, 1::2]))
    oe = _pad_lanes(xo[:, :, :_HO, 0::2])
    oo = _pad_lanes(xo[:, :, :_HO, 1::2])

    out = pl.pallas_call(
        _conv_body,
        grid=(_B, _NRB, _NOCB),
        in_specs=[
            pl.BlockSpec((_K, _K, _OCB, _IN_CH), lambda b, r, o: (0, 0, o, 0)),
            pl.BlockSpec((None, _IN_CH, _HALO, _LANE), lambda b, r, o: (b, 0, r, 0)),
            pl.BlockSpec((None, _IN_CH, _HALO, _LANE), lambda b, r, o: (b, 0, r, 0)),
            pl.BlockSpec((None, _IN_CH, _RB, _LANE), lambda b, r, o: (b, 0, r, 0)),
            pl.BlockSpec((None, _IN_CH, _RB, _LANE), lambda b, r, o: (b, 0, r, 0)),
        ],
        out_specs=pl.BlockSpec((None, _OCB, _RB, _LANE), lambda b, r, o: (b, o, r, 0)),
        out_shape=jax.ShapeDtypeStruct((_B, _OUT_CH, _HO, _LANE), jnp.float32),
        compiler_params=pltpu.CompilerParams(
            dimension_semantics=("parallel", "parallel", "arbitrary")),
    )(wt, ee, eo, oe, oo)
    return out[:, :, :, :_WO]


def _active_mask(x, weight, hash_a, hash_b):
    w_flat = weight.reshape(_OUT_CH, -1)
    denom = jnp.linalg.norm(w_flat, axis=1).max()
    w_u = _U * w_flat / denom
    norms = jnp.linalg.norm(w_u, axis=1, keepdims=True)
    powers = jnp.concatenate([norms ** (2 ** (i + 1)) for i in range(_M)], axis=1)
    halves = jnp.full((_OUT_CH, _M), 0.5, dtype=w_u.dtype)
    w_pq = jnp.concatenate([w_u, powers, halves], axis=1)
    k_proj = w_pq @ hash_a.T + hash_b[None, :]
    k_idx = jnp.abs(jnp.mod(jnp.floor(k_proj / _R).astype(jnp.int32), _TABLE))

    x_u = _U * x / denom
    q_chan = jnp.full((_B, 1, _H, _W), 0.5, dtype=x.dtype)
    p_chan = jnp.broadcast_to(
        (jnp.linalg.norm(x_u.reshape(_B, -1), axis=1) ** 2).reshape(_B, 1, 1, 1),
        (_B, 1, _H, _W)).astype(x.dtype)
    x_aug = jnp.concatenate([x_u, q_chan, p_chan], axis=1)
    hk = hash_a.reshape(_NH, _IN_CH + 2, _K, _K)
    dotted = jax.lax.conv_general_dilated(
        x_aug, hk, window_strides=(_STRIDE, _STRIDE),
        padding=((_PAD, _PAD), (_PAD, _PAD)),
        rhs_dilation=(1, 1),
        dimension_numbers=('NCHW', 'OIHW', 'NCHW'))
    bucket = jnp.abs(jnp.mod(
        jnp.floor((dotted + hash_b.reshape(1, -1, 1, 1)) / _R).astype(jnp.int32),
        _TABLE))
    counts = jnp.stack([jnp.bincount(bucket[:, h].ravel(), length=_TABLE)
                        for h in range(_NH)])
    best = jnp.argmax(counts, axis=1)
    return jnp.any(k_idx == best[None, :], axis=1)


def kernel(x, weight, hash_a, hash_b):
    active = _active_mask(x, weight, hash_a, hash_b)
    scale = jnp.asarray(_NH / _TABLE, dtype=x.dtype)
    w_eff = weight * (active.astype(x.dtype) * scale)[:, None, None, None]
    wt = jnp.transpose(w_eff, (2, 3, 0, 1))
    return _conv_pallas(wt, x)
